# baseline (speedup 1.0000x reference)
import functools

import jax
import jax.numpy as jnp
from jax import lax
from jax.experimental import pallas as pl
from jax.experimental.pallas import tpu as pltpu
from jax.experimental.pallas import tpu_sc as plsc

L = 16
NC = 2
NS = 16
NW = NC * NS
B = 16384
F = 64
N_ROWS = 1000000
SPLIT = 512000
BPW = B // NW
IDX_CHUNK = 128
NCHUNK = BPW // IDX_CHUNK
HALF = BPW // 2

N_CBLK = N_ROWS // IDX_CHUNK + (1 if N_ROWS % IDX_CHUNK else 0)
GRID_T = SPLIT // IDX_CHUNK


def _transpose_body(u0_ref, u1_ref, b0_ref, b1_ref, uo_ref, bo_ref):
    uo_ref[...] = jnp.concatenate([u0_ref[...].T, u1_ref[...].T], axis=1)
    bo_ref[...] = jnp.concatenate([b0_ref[...].T, b1_ref[...].T], axis=1)


_transpose = pl.pallas_call(
    _transpose_body,
    grid=(GRID_T,),
    in_specs=[
        pl.BlockSpec((F, IDX_CHUNK), lambda g: (0, g)),
        pl.BlockSpec((F, IDX_CHUNK),
                     lambda g: (0, jnp.minimum(GRID_T + g, N_CBLK - 1))),
        pl.BlockSpec((F, IDX_CHUNK), lambda g: (0, g)),
        pl.BlockSpec((F, IDX_CHUNK),
                     lambda g: (0, jnp.minimum(GRID_T + g, N_CBLK - 1))),
    ],
    out_specs=[
        pl.BlockSpec((IDX_CHUNK, 2 * F), lambda g: (g, 0)),
        pl.BlockSpec((IDX_CHUNK, 2 * F), lambda g: (g, 0)),
    ],
    out_shape=[
        jax.ShapeDtypeStruct((SPLIT, 2 * F), jnp.float32),
        jax.ShapeDtypeStruct((SPLIT, 2 * F), jnp.float32),
    ],
)


def _make_sc_kernel():
    mesh = plsc.VectorSubcoreMesh(core_axis_name="c", subcore_axis_name="s")

    @functools.partial(
        pl.kernel,
        mesh=mesh,
        compiler_params=pltpu.CompilerParams(
            needs_layout_passes=False, use_tc_tiling_on_sc=False),
        out_type=jax.ShapeDtypeStruct((B,), jnp.float32),
        scratch_types=[
            pltpu.VMEM((NCHUNK, IDX_CHUNK), jnp.int32),
            pltpu.VMEM((NCHUNK, IDX_CHUNK), jnp.int32),
            pltpu.VMEM((BPW,), jnp.int32),
            pltpu.VMEM((BPW,), jnp.int32),
            pltpu.VMEM((HALF, 2 * F), jnp.float32),
            pltpu.VMEM((HALF, 2 * F), jnp.float32),
            pltpu.VMEM((BPW,), jnp.float32),
            pltpu.SemaphoreType.DMA,
        ],
    )
    def kern(user_hbm, book_hbm, upar_hbm, bpar_hbm, uf_hbm, bf_hbm, out_hbm,
             uidx_v, bidx_v, upar_v, bpar_v, urows_v, brows_v, out_v, sem):
        wid = lax.axis_index("s") * NC + lax.axis_index("c")
        base = wid * BPW

        pltpu.sync_copy(user_hbm.at[wid], uidx_v)
        pltpu.sync_copy(book_hbm.at[wid], bidx_v)
        pltpu.sync_copy(upar_hbm.at[pl.ds(base, BPW)], upar_v)
        pltpu.sync_copy(bpar_hbm.at[pl.ds(base, BPW)], bpar_v)

        rows16 = lax.iota(jnp.int32, L)

        for h in range(2):
            copies = []
            for i in range(HALF // IDX_CHUNK):
                c = h * (HALF // IDX_CHUNK) + i
                copies.append(pltpu.async_copy(
                    uf_hbm.at[uidx_v.at[c]],
                    urows_v.at[pl.ds(i * IDX_CHUNK, IDX_CHUNK)], sem))
                copies.append(pltpu.async_copy(
                    bf_hbm.at[bidx_v.at[c]],
                    brows_v.at[pl.ds(i * IDX_CHUNK, IDX_CHUNK)], sem))
            for cp in copies:
                cp.wait()

            def group_body(g, carry, h=h):
                lrow0 = g * L
                ridx = lrow0 + rows16
                ucol = upar_v[pl.ds(h * HALF + lrow0, L)]
                bcol = bpar_v[pl.ds(h * HALF + lrow0, L)]

                def feat_body(f, acc):
                    u = plsc.load_gather(urows_v, [ridx, ucol + f])
                    b = plsc.load_gather(brows_v, [ridx, bcol + f])
                    return acc + u * b

                acc = lax.fori_loop(0, F, feat_body,
                                    jnp.zeros((L,), jnp.float32), unroll=8)
                out_v[pl.ds(h * HALF + lrow0, L)] = acc
                return carry

            lax.fori_loop(0, HALF // L, group_body, 0)

        pltpu.sync_copy(out_v, out_hbm.at[pl.ds(base, BPW)])

    return kern


_sc_kernel = _make_sc_kernel()


@jax.jit
def kernel(user, book, user_factors, book_factors):
    user_i = user.astype(jnp.int32)
    book_i = book.astype(jnp.int32)
    user_sr = jnp.where(user_i >= SPLIT, user_i - SPLIT, user_i)
    book_sr = jnp.where(book_i >= SPLIT, book_i - SPLIT, book_i)
    upar = jnp.where(user_i >= SPLIT, F, 0).astype(jnp.int32)
    bpar = jnp.where(book_i >= SPLIT, F, 0).astype(jnp.int32)
    uft = user_factors.T
    bft = book_factors.T
    uf2, bf2 = _transpose(uft, uft, bft, bft)
    return _sc_kernel(
        user_sr.reshape(NW, NCHUNK, IDX_CHUNK),
        book_sr.reshape(NW, NCHUNK, IDX_CHUNK),
        upar, bpar, uf2, bf2)


# speedup vs baseline: 3.4555x; 3.4555x over previous
import functools

import jax
import jax.numpy as jnp
from jax import lax
from jax.experimental import pallas as pl
from jax.experimental.pallas import tpu as pltpu
from jax.experimental.pallas import tpu_sc as plsc

L = 16
NC = 2
NS = 16
NW = NC * NS
B = 16384
F = 64
N_ROWS = 1000000
SPLIT = 512000
BPW = B // NW
IDX_CHUNK = 128
NCHUNK = BPW // IDX_CHUNK
HALF = BPW // 2

TBLK = 1024
N_CBLK = N_ROWS // TBLK + (1 if N_ROWS % TBLK else 0)
GRID_T = SPLIT // TBLK


def _transpose_body(u0_ref, u1_ref, b0_ref, b1_ref, uo_ref, bo_ref):
    uo_ref[...] = jnp.concatenate([u0_ref[...].T, u1_ref[...].T], axis=1)
    bo_ref[...] = jnp.concatenate([b0_ref[...].T, b1_ref[...].T], axis=1)


_transpose = pl.pallas_call(
    _transpose_body,
    grid=(GRID_T,),
    in_specs=[
        pl.BlockSpec((F, TBLK), lambda g: (0, g)),
        pl.BlockSpec((F, TBLK),
                     lambda g: (0, jnp.minimum(GRID_T + g, N_CBLK - 1))),
        pl.BlockSpec((F, TBLK), lambda g: (0, g)),
        pl.BlockSpec((F, TBLK),
                     lambda g: (0, jnp.minimum(GRID_T + g, N_CBLK - 1))),
    ],
    out_specs=[
        pl.BlockSpec((TBLK, 2 * F), lambda g: (g, 0)),
        pl.BlockSpec((TBLK, 2 * F), lambda g: (g, 0)),
    ],
    out_shape=[
        jax.ShapeDtypeStruct((SPLIT, 2 * F), jnp.float32),
        jax.ShapeDtypeStruct((SPLIT, 2 * F), jnp.float32),
    ],
)


def _make_sc_kernel():
    mesh = plsc.VectorSubcoreMesh(core_axis_name="c", subcore_axis_name="s")

    @functools.partial(
        pl.kernel,
        mesh=mesh,
        compiler_params=pltpu.CompilerParams(
            needs_layout_passes=False, use_tc_tiling_on_sc=False),
        out_type=jax.ShapeDtypeStruct((B,), jnp.float32),
        scratch_types=[
            pltpu.VMEM((NCHUNK, IDX_CHUNK), jnp.int32),
            pltpu.VMEM((NCHUNK, IDX_CHUNK), jnp.int32),
            pltpu.VMEM((BPW,), jnp.int32),
            pltpu.VMEM((BPW,), jnp.int32),
            pltpu.VMEM((HALF, 2 * F), jnp.float32),
            pltpu.VMEM((HALF, 2 * F), jnp.float32),
            pltpu.VMEM((BPW,), jnp.float32),
            pltpu.SemaphoreType.DMA,
        ],
    )
    def kern(user_hbm, book_hbm, upar_hbm, bpar_hbm, uf_hbm, bf_hbm, out_hbm,
             uidx_v, bidx_v, upar_v, bpar_v, urows_v, brows_v, out_v, sem):
        wid = lax.axis_index("s") * NC + lax.axis_index("c")
        base = wid * BPW

        pltpu.sync_copy(user_hbm.at[wid], uidx_v)
        pltpu.sync_copy(book_hbm.at[wid], bidx_v)
        pltpu.sync_copy(upar_hbm.at[pl.ds(base, BPW)], upar_v)
        pltpu.sync_copy(bpar_hbm.at[pl.ds(base, BPW)], bpar_v)

        rows16 = lax.iota(jnp.int32, L)

        for h in range(2):
            copies = []
            for i in range(HALF // IDX_CHUNK):
                c = h * (HALF // IDX_CHUNK) + i
                copies.append(pltpu.async_copy(
                    uf_hbm.at[uidx_v.at[c]],
                    urows_v.at[pl.ds(i * IDX_CHUNK, IDX_CHUNK)], sem))
                copies.append(pltpu.async_copy(
                    bf_hbm.at[bidx_v.at[c]],
                    brows_v.at[pl.ds(i * IDX_CHUNK, IDX_CHUNK)], sem))
            for cp in copies:
                cp.wait()

            def group_body(g, carry, h=h):
                lrow0 = g * L
                ridx = lrow0 + rows16
                ucol = upar_v[pl.ds(h * HALF + lrow0, L)]
                bcol = bpar_v[pl.ds(h * HALF + lrow0, L)]

                def feat_body(f, acc):
                    u = plsc.load_gather(urows_v, [ridx, ucol + f])
                    b = plsc.load_gather(brows_v, [ridx, bcol + f])
                    return acc + u * b

                acc = lax.fori_loop(0, F, feat_body,
                                    jnp.zeros((L,), jnp.float32), unroll=8)
                out_v[pl.ds(h * HALF + lrow0, L)] = acc
                return carry

            lax.fori_loop(0, HALF // L, group_body, 0)

        pltpu.sync_copy(out_v, out_hbm.at[pl.ds(base, BPW)])

    return kern


_sc_kernel = _make_sc_kernel()


@jax.jit
def kernel(user, book, user_factors, book_factors):
    user_i = user.astype(jnp.int32)
    book_i = book.astype(jnp.int32)
    user_sr = jnp.where(user_i >= SPLIT, user_i - SPLIT, user_i)
    book_sr = jnp.where(book_i >= SPLIT, book_i - SPLIT, book_i)
    upar = jnp.where(user_i >= SPLIT, F, 0).astype(jnp.int32)
    bpar = jnp.where(book_i >= SPLIT, F, 0).astype(jnp.int32)
    uft = user_factors.T
    bft = book_factors.T
    uf2, bf2 = _transpose(uft, uft, bft, bft)
    return _sc_kernel(
        user_sr.reshape(NW, NCHUNK, IDX_CHUNK),
        book_sr.reshape(NW, NCHUNK, IDX_CHUNK),
        upar, bpar, uf2, bf2)


# speedup vs baseline: 3.4555x; 1.0000x over previous
import functools

import jax
import jax.numpy as jnp
from jax import lax
from jax.experimental import pallas as pl
from jax.experimental.pallas import tpu as pltpu
from jax.experimental.pallas import tpu_sc as plsc

L = 16
NC = 2
NS = 16
NW = NC * NS
B = 16384
F = 64
N_ROWS = 1000000
SPLIT = 512000
BPW = B // NW
IDX_CHUNK = 128
NCHUNK = BPW // IDX_CHUNK
HALF = BPW // 2

TBLK = 1024
N_CBLK = N_ROWS // TBLK + (1 if N_ROWS % TBLK else 0)
GRID_T = SPLIT // TBLK


def _transpose_body(u0_ref, u1_ref, b0_ref, b1_ref, uo_ref, bo_ref):
    eye = (lax.broadcasted_iota(jnp.int32, (F, F), 0)
           == lax.broadcasted_iota(jnp.int32, (F, F), 1)).astype(jnp.float32)

    def tr(ref):
        return lax.dot_general(ref[...], eye, (((0,), (0,)), ((), ())),
                               preferred_element_type=jnp.float32)

    uo_ref[...] = jnp.concatenate([tr(u0_ref), tr(u1_ref)], axis=1)
    bo_ref[...] = jnp.concatenate([tr(b0_ref), tr(b1_ref)], axis=1)


_transpose = pl.pallas_call(
    _transpose_body,
    grid=(GRID_T,),
    in_specs=[
        pl.BlockSpec((F, TBLK), lambda g: (0, g)),
        pl.BlockSpec((F, TBLK),
                     lambda g: (0, jnp.minimum(GRID_T + g, N_CBLK - 1))),
        pl.BlockSpec((F, TBLK), lambda g: (0, g)),
        pl.BlockSpec((F, TBLK),
                     lambda g: (0, jnp.minimum(GRID_T + g, N_CBLK - 1))),
    ],
    out_specs=[
        pl.BlockSpec((TBLK, 2 * F), lambda g: (g, 0)),
        pl.BlockSpec((TBLK, 2 * F), lambda g: (g, 0)),
    ],
    out_shape=[
        jax.ShapeDtypeStruct((SPLIT, 2 * F), jnp.float32),
        jax.ShapeDtypeStruct((SPLIT, 2 * F), jnp.float32),
    ],
)


def _make_sc_kernel():
    mesh = plsc.VectorSubcoreMesh(core_axis_name="c", subcore_axis_name="s")

    @functools.partial(
        pl.kernel,
        mesh=mesh,
        compiler_params=pltpu.CompilerParams(
            needs_layout_passes=False, use_tc_tiling_on_sc=False),
        out_type=jax.ShapeDtypeStruct((B,), jnp.float32),
        scratch_types=[
            pltpu.VMEM((NCHUNK, IDX_CHUNK), jnp.int32),
            pltpu.VMEM((NCHUNK, IDX_CHUNK), jnp.int32),
            pltpu.VMEM((BPW,), jnp.int32),
            pltpu.VMEM((BPW,), jnp.int32),
            pltpu.VMEM((HALF, 2 * F), jnp.float32),
            pltpu.VMEM((HALF, 2 * F), jnp.float32),
            pltpu.VMEM((BPW,), jnp.float32),
            pltpu.SemaphoreType.DMA,
        ],
    )
    def kern(user_hbm, book_hbm, upar_hbm, bpar_hbm, uf_hbm, bf_hbm, out_hbm,
             uidx_v, bidx_v, upar_v, bpar_v, urows_v, brows_v, out_v, sem):
        wid = lax.axis_index("s") * NC + lax.axis_index("c")
        base = wid * BPW

        pltpu.sync_copy(user_hbm.at[wid], uidx_v)
        pltpu.sync_copy(book_hbm.at[wid], bidx_v)
        pltpu.sync_copy(upar_hbm.at[pl.ds(base, BPW)], upar_v)
        pltpu.sync_copy(bpar_hbm.at[pl.ds(base, BPW)], bpar_v)

        rows16 = lax.iota(jnp.int32, L)

        for h in range(2):
            copies = []
            for i in range(HALF // IDX_CHUNK):
                c = h * (HALF // IDX_CHUNK) + i
                copies.append(pltpu.async_copy(
                    uf_hbm.at[uidx_v.at[c]],
                    urows_v.at[pl.ds(i * IDX_CHUNK, IDX_CHUNK)], sem))
                copies.append(pltpu.async_copy(
                    bf_hbm.at[bidx_v.at[c]],
                    brows_v.at[pl.ds(i * IDX_CHUNK, IDX_CHUNK)], sem))
            for cp in copies:
                cp.wait()

            def group_body(g, carry, h=h):
                lrow0 = g * L
                ridx = lrow0 + rows16
                ucol = upar_v[pl.ds(h * HALF + lrow0, L)]
                bcol = bpar_v[pl.ds(h * HALF + lrow0, L)]

                def feat_body(f, acc):
                    u = plsc.load_gather(urows_v, [ridx, ucol + f])
                    b = plsc.load_gather(brows_v, [ridx, bcol + f])
                    return acc + u * b

                acc = lax.fori_loop(0, F, feat_body,
                                    jnp.zeros((L,), jnp.float32), unroll=8)
                out_v[pl.ds(h * HALF + lrow0, L)] = acc
                return carry

            lax.fori_loop(0, HALF // L, group_body, 0)

        pltpu.sync_copy(out_v, out_hbm.at[pl.ds(base, BPW)])

    return kern


_sc_kernel = _make_sc_kernel()


@jax.jit
def kernel(user, book, user_factors, book_factors):
    user_i = user.astype(jnp.int32)
    book_i = book.astype(jnp.int32)
    user_sr = jnp.where(user_i >= SPLIT, user_i - SPLIT, user_i)
    book_sr = jnp.where(book_i >= SPLIT, book_i - SPLIT, book_i)
    upar = jnp.where(user_i >= SPLIT, F, 0).astype(jnp.int32)
    bpar = jnp.where(book_i >= SPLIT, F, 0).astype(jnp.int32)
    uft = user_factors.T
    bft = book_factors.T
    uf2, bf2 = _transpose(uft, uft, bft, bft)
    return _sc_kernel(
        user_sr.reshape(NW, NCHUNK, IDX_CHUNK),
        book_sr.reshape(NW, NCHUNK, IDX_CHUNK),
        upar, bpar, uf2, bf2)


# speedup vs baseline: 4.3848x; 1.2689x over previous
import functools

import jax
import jax.numpy as jnp
from jax import lax
from jax.experimental import pallas as pl
from jax.experimental.pallas import tpu as pltpu
from jax.experimental.pallas import tpu_sc as plsc

L = 16
NC = 2
NS = 16
NW = NC * NS
B = 16384
F = 64
N_ROWS = 1000000
SPLIT = 512000
BPW = B // NW
IDX_CHUNK = 128
NCHUNK = BPW // IDX_CHUNK
HALF = BPW // 2

TBLK = 2048
N_CBLK = N_ROWS // TBLK + (1 if N_ROWS % TBLK else 0)
GRID_T = SPLIT // TBLK


def _transpose_body(u0_ref, u1_ref, b0_ref, b1_ref, uo_ref, bo_ref):
    eye = (lax.broadcasted_iota(jnp.int32, (F, F), 0)
           == lax.broadcasted_iota(jnp.int32, (F, F), 1)).astype(jnp.float32)

    def tr(ref):
        return lax.dot_general(ref[...], eye, (((0,), (0,)), ((), ())),
                               preferred_element_type=jnp.float32)

    uo_ref[...] = jnp.concatenate([tr(u0_ref), tr(u1_ref)], axis=1)
    bo_ref[...] = jnp.concatenate([tr(b0_ref), tr(b1_ref)], axis=1)


_transpose = pl.pallas_call(
    _transpose_body,
    grid=(GRID_T,),
    in_specs=[
        pl.BlockSpec((F, TBLK), lambda g: (0, g)),
        pl.BlockSpec((F, TBLK),
                     lambda g: (0, jnp.minimum(GRID_T + g, N_CBLK - 1))),
        pl.BlockSpec((F, TBLK), lambda g: (0, g)),
        pl.BlockSpec((F, TBLK),
                     lambda g: (0, jnp.minimum(GRID_T + g, N_CBLK - 1))),
    ],
    out_specs=[
        pl.BlockSpec((TBLK, 2 * F), lambda g: (g, 0)),
        pl.BlockSpec((TBLK, 2 * F), lambda g: (g, 0)),
    ],
    out_shape=[
        jax.ShapeDtypeStruct((SPLIT, 2 * F), jnp.float32),
        jax.ShapeDtypeStruct((SPLIT, 2 * F), jnp.float32),
    ],
)


def _make_sc_kernel():
    mesh = plsc.VectorSubcoreMesh(core_axis_name="c", subcore_axis_name="s")

    @functools.partial(
        pl.kernel,
        mesh=mesh,
        compiler_params=pltpu.CompilerParams(
            needs_layout_passes=False, use_tc_tiling_on_sc=False),
        out_type=jax.ShapeDtypeStruct((B,), jnp.float32),
        scratch_types=[
            pltpu.VMEM((NCHUNK, IDX_CHUNK), jnp.int32),
            pltpu.VMEM((NCHUNK, IDX_CHUNK), jnp.int32),
            pltpu.VMEM((BPW,), jnp.int32),
            pltpu.VMEM((BPW,), jnp.int32),
            pltpu.VMEM((HALF, 2 * F), jnp.float32),
            pltpu.VMEM((HALF, 2 * F), jnp.float32),
            pltpu.VMEM((BPW,), jnp.float32),
            pltpu.SemaphoreType.DMA,
        ],
    )
    def kern(user_hbm, book_hbm, upar_hbm, bpar_hbm, uf_hbm, bf_hbm, out_hbm,
             uidx_v, bidx_v, upar_v, bpar_v, urows_v, brows_v, out_v, sem):
        wid = lax.axis_index("s") * NC + lax.axis_index("c")
        base = wid * BPW

        pltpu.sync_copy(user_hbm.at[wid], uidx_v)
        pltpu.sync_copy(book_hbm.at[wid], bidx_v)
        pltpu.sync_copy(upar_hbm.at[pl.ds(base, BPW)], upar_v)
        pltpu.sync_copy(bpar_hbm.at[pl.ds(base, BPW)], bpar_v)

        rows16 = lax.iota(jnp.int32, L)

        for h in range(2):
            copies = []
            for i in range(HALF // IDX_CHUNK):
                c = h * (HALF // IDX_CHUNK) + i
                copies.append(pltpu.async_copy(
                    uf_hbm.at[uidx_v.at[c]],
                    urows_v.at[pl.ds(i * IDX_CHUNK, IDX_CHUNK)], sem))
                copies.append(pltpu.async_copy(
                    bf_hbm.at[bidx_v.at[c]],
                    brows_v.at[pl.ds(i * IDX_CHUNK, IDX_CHUNK)], sem))
            for cp in copies:
                cp.wait()

            def group_body(g, carry, h=h):
                lrow0 = g * L
                ridx = lrow0 + rows16
                ucol = upar_v[pl.ds(h * HALF + lrow0, L)]
                bcol = bpar_v[pl.ds(h * HALF + lrow0, L)]

                def feat_body(f, acc):
                    u = plsc.load_gather(urows_v, [ridx, ucol + f])
                    b = plsc.load_gather(brows_v, [ridx, bcol + f])
                    return acc + u * b

                acc = lax.fori_loop(0, F, feat_body,
                                    jnp.zeros((L,), jnp.float32), unroll=8)
                out_v[pl.ds(h * HALF + lrow0, L)] = acc
                return carry

            lax.fori_loop(0, HALF // L, group_body, 0)

        pltpu.sync_copy(out_v, out_hbm.at[pl.ds(base, BPW)])

    return kern


_sc_kernel = _make_sc_kernel()


@jax.jit
def kernel(user, book, user_factors, book_factors):
    user_i = user.astype(jnp.int32)
    book_i = book.astype(jnp.int32)
    user_sr = jnp.where(user_i >= SPLIT, user_i - SPLIT, user_i)
    book_sr = jnp.where(book_i >= SPLIT, book_i - SPLIT, book_i)
    upar = jnp.where(user_i >= SPLIT, F, 0).astype(jnp.int32)
    bpar = jnp.where(book_i >= SPLIT, F, 0).astype(jnp.int32)
    uft = user_factors.T
    bft = book_factors.T
    uf2, bf2 = _transpose(uft, uft, bft, bft)
    return _sc_kernel(
        user_sr.reshape(NW, NCHUNK, IDX_CHUNK),
        book_sr.reshape(NW, NCHUNK, IDX_CHUNK),
        upar, bpar, uf2, bf2)


# speedup vs baseline: 5.0786x; 1.1582x over previous
import functools

import jax
import jax.numpy as jnp
from jax import lax
from jax.experimental import pallas as pl
from jax.experimental.pallas import tpu as pltpu
from jax.experimental.pallas import tpu_sc as plsc

L = 16
NC = 2
NS = 16
NW = NC * NS
B = 16384
F = 64
N_ROWS = 1000000
SPLIT = 512000
BPW = B // NW
IDX_CHUNK = 128
NCHUNK = BPW // IDX_CHUNK
HALF = BPW // 2

TBLK = 4096
N_CBLK = N_ROWS // TBLK + (1 if N_ROWS % TBLK else 0)
GRID_T = SPLIT // TBLK


def _transpose_body(u0_ref, u1_ref, b0_ref, b1_ref, uo_ref, bo_ref):
    eye = (lax.broadcasted_iota(jnp.int32, (F, F), 0)
           == lax.broadcasted_iota(jnp.int32, (F, F), 1)).astype(jnp.float32)

    def tr(ref):
        return lax.dot_general(ref[...], eye, (((0,), (0,)), ((), ())),
                               preferred_element_type=jnp.float32)

    uo_ref[...] = jnp.concatenate([tr(u0_ref), tr(u1_ref)], axis=1)
    bo_ref[...] = jnp.concatenate([tr(b0_ref), tr(b1_ref)], axis=1)


_transpose = pl.pallas_call(
    _transpose_body,
    grid=(GRID_T,),
    in_specs=[
        pl.BlockSpec((F, TBLK), lambda g: (0, g)),
        pl.BlockSpec((F, TBLK),
                     lambda g: (0, jnp.minimum(GRID_T + g, N_CBLK - 1))),
        pl.BlockSpec((F, TBLK), lambda g: (0, g)),
        pl.BlockSpec((F, TBLK),
                     lambda g: (0, jnp.minimum(GRID_T + g, N_CBLK - 1))),
    ],
    out_specs=[
        pl.BlockSpec((TBLK, 2 * F), lambda g: (g, 0)),
        pl.BlockSpec((TBLK, 2 * F), lambda g: (g, 0)),
    ],
    out_shape=[
        jax.ShapeDtypeStruct((SPLIT, 2 * F), jnp.float32),
        jax.ShapeDtypeStruct((SPLIT, 2 * F), jnp.float32),
    ],
)


def _make_sc_kernel():
    mesh = plsc.VectorSubcoreMesh(core_axis_name="c", subcore_axis_name="s")

    @functools.partial(
        pl.kernel,
        mesh=mesh,
        compiler_params=pltpu.CompilerParams(
            needs_layout_passes=False, use_tc_tiling_on_sc=False),
        out_type=jax.ShapeDtypeStruct((B,), jnp.float32),
        scratch_types=[
            pltpu.VMEM((NCHUNK, IDX_CHUNK), jnp.int32),
            pltpu.VMEM((NCHUNK, IDX_CHUNK), jnp.int32),
            pltpu.VMEM((BPW,), jnp.int32),
            pltpu.VMEM((BPW,), jnp.int32),
            pltpu.VMEM((HALF, 2 * F), jnp.float32),
            pltpu.VMEM((HALF, 2 * F), jnp.float32),
            pltpu.VMEM((BPW,), jnp.float32),
            pltpu.SemaphoreType.DMA,
        ],
    )
    def kern(user_hbm, book_hbm, upar_hbm, bpar_hbm, uf_hbm, bf_hbm, out_hbm,
             uidx_v, bidx_v, upar_v, bpar_v, urows_v, brows_v, out_v, sem):
        wid = lax.axis_index("s") * NC + lax.axis_index("c")
        base = wid * BPW

        pltpu.sync_copy(user_hbm.at[wid], uidx_v)
        pltpu.sync_copy(book_hbm.at[wid], bidx_v)
        pltpu.sync_copy(upar_hbm.at[pl.ds(base, BPW)], upar_v)
        pltpu.sync_copy(bpar_hbm.at[pl.ds(base, BPW)], bpar_v)

        rows16 = lax.iota(jnp.int32, L)

        for h in range(2):
            copies = []
            for i in range(HALF // IDX_CHUNK):
                c = h * (HALF // IDX_CHUNK) + i
                copies.append(pltpu.async_copy(
                    uf_hbm.at[uidx_v.at[c]],
                    urows_v.at[pl.ds(i * IDX_CHUNK, IDX_CHUNK)], sem))
                copies.append(pltpu.async_copy(
                    bf_hbm.at[bidx_v.at[c]],
                    brows_v.at[pl.ds(i * IDX_CHUNK, IDX_CHUNK)], sem))
            for cp in copies:
                cp.wait()

            def group_body(g, carry, h=h):
                lrow0 = g * L
                ridx = lrow0 + rows16
                ucol = upar_v[pl.ds(h * HALF + lrow0, L)]
                bcol = bpar_v[pl.ds(h * HALF + lrow0, L)]

                def feat_body(f, acc):
                    u = plsc.load_gather(urows_v, [ridx, ucol + f])
                    b = plsc.load_gather(brows_v, [ridx, bcol + f])
                    return acc + u * b

                acc = lax.fori_loop(0, F, feat_body,
                                    jnp.zeros((L,), jnp.float32), unroll=8)
                out_v[pl.ds(h * HALF + lrow0, L)] = acc
                return carry

            lax.fori_loop(0, HALF // L, group_body, 0)

        pltpu.sync_copy(out_v, out_hbm.at[pl.ds(base, BPW)])

    return kern


_sc_kernel = _make_sc_kernel()


@jax.jit
def kernel(user, book, user_factors, book_factors):
    user_i = user.astype(jnp.int32)
    book_i = book.astype(jnp.int32)
    user_sr = jnp.where(user_i >= SPLIT, user_i - SPLIT, user_i)
    book_sr = jnp.where(book_i >= SPLIT, book_i - SPLIT, book_i)
    upar = jnp.where(user_i >= SPLIT, F, 0).astype(jnp.int32)
    bpar = jnp.where(book_i >= SPLIT, F, 0).astype(jnp.int32)
    uft = user_factors.T
    bft = book_factors.T
    uf2, bf2 = _transpose(uft, uft, bft, bft)
    return _sc_kernel(
        user_sr.reshape(NW, NCHUNK, IDX_CHUNK),
        book_sr.reshape(NW, NCHUNK, IDX_CHUNK),
        upar, bpar, uf2, bf2)


# speedup vs baseline: 5.7422x; 1.1307x over previous
import functools

import jax
import jax.numpy as jnp
from jax import lax
from jax.experimental import pallas as pl
from jax.experimental.pallas import tpu as pltpu
from jax.experimental.pallas import tpu_sc as plsc

L = 16
NC = 2
NS = 16
NW = NC * NS
B = 16384
F = 64
N_ROWS = 1000000
QUART = 256000
NREC = QUART
BPW = B // NW
IDX_CHUNK = 128
NCHUNK = BPW // IDX_CHUNK
HALF = BPW // 2

TBLK = 2048
N_CBLK = N_ROWS // TBLK + (1 if N_ROWS % TBLK else 0)
GRID_T = QUART // TBLK
QBLK = QUART // TBLK


def _transpose_body(u0_ref, u1_ref, u2_ref, u3_ref,
                    b0_ref, b1_ref, b2_ref, b3_ref, uo_ref, bo_ref):
    eye = (lax.broadcasted_iota(jnp.int32, (F, F), 0)
           == lax.broadcasted_iota(jnp.int32, (F, F), 1)).astype(jnp.float32)

    def tr(ref):
        return lax.dot_general(ref[...], eye, (((0,), (0,)), ((), ())),
                               preferred_element_type=jnp.float32)

    def pack2(lo_f32, hi_f32):
        lo = lax.bitcast_convert_type(
            lo_f32.astype(jnp.bfloat16), jnp.uint16).astype(jnp.uint32)
        hi = lax.bitcast_convert_type(
            hi_f32.astype(jnp.bfloat16), jnp.uint16).astype(jnp.uint32)
        return lax.bitcast_convert_type(lo | (hi << 16), jnp.float32)

    uo_ref[...] = jnp.concatenate(
        [pack2(tr(u0_ref), tr(u1_ref)), pack2(tr(u2_ref), tr(u3_ref))],
        axis=1)
    bo_ref[...] = jnp.concatenate(
        [pack2(tr(b0_ref), tr(b1_ref)), pack2(tr(b2_ref), tr(b3_ref))],
        axis=1)


def _window_spec(q):
    return pl.BlockSpec(
        (F, TBLK), lambda g, q=q: (0, jnp.minimum(q * QBLK + g, N_CBLK - 1)))


_transpose = pl.pallas_call(
    _transpose_body,
    grid=(GRID_T,),
    in_specs=[_window_spec(q) for q in range(4)] * 2,
    out_specs=[
        pl.BlockSpec((TBLK, 2 * F), lambda g: (g, 0)),
        pl.BlockSpec((TBLK, 2 * F), lambda g: (g, 0)),
    ],
    out_shape=[
        jax.ShapeDtypeStruct((NREC, 2 * F), jnp.float32),
        jax.ShapeDtypeStruct((NREC, 2 * F), jnp.float32),
    ],
)


def _make_sc_kernel():
    mesh = plsc.VectorSubcoreMesh(core_axis_name="c", subcore_axis_name="s")

    @functools.partial(
        pl.kernel,
        mesh=mesh,
        compiler_params=pltpu.CompilerParams(
            needs_layout_passes=False, use_tc_tiling_on_sc=False),
        out_type=jax.ShapeDtypeStruct((B,), jnp.float32),
        scratch_types=[
            pltpu.VMEM((NCHUNK, IDX_CHUNK), jnp.int32),
            pltpu.VMEM((NCHUNK, IDX_CHUNK), jnp.int32),
            pltpu.VMEM((BPW,), jnp.int32),
            pltpu.VMEM((BPW,), jnp.int32),
            pltpu.VMEM((BPW,), jnp.int32),
            pltpu.VMEM((BPW,), jnp.int32),
            pltpu.VMEM((HALF, 2 * F), jnp.float32),
            pltpu.VMEM((HALF, 2 * F), jnp.float32),
            pltpu.VMEM((BPW,), jnp.float32),
            pltpu.SemaphoreType.DMA,
        ],
    )
    def kern(user_hbm, book_hbm, ucol_hbm, bcol_hbm, upar_hbm, bpar_hbm,
             uf_hbm, bf_hbm, out_hbm,
             uidx_v, bidx_v, ucol_v, bcol_v, upar_v, bpar_v,
             urows_v, brows_v, out_v, sem):
        wid = lax.axis_index("s") * NC + lax.axis_index("c")
        base = wid * BPW

        pltpu.sync_copy(user_hbm.at[wid], uidx_v)
        pltpu.sync_copy(book_hbm.at[wid], bidx_v)
        pltpu.sync_copy(ucol_hbm.at[pl.ds(base, BPW)], ucol_v)
        pltpu.sync_copy(bcol_hbm.at[pl.ds(base, BPW)], bcol_v)
        pltpu.sync_copy(upar_hbm.at[pl.ds(base, BPW)], upar_v)
        pltpu.sync_copy(bpar_hbm.at[pl.ds(base, BPW)], bpar_v)

        rows16 = lax.iota(jnp.int32, L)
        zero16 = jnp.zeros((L,), jnp.int32)

        for h in range(2):
            copies = []
            for i in range(HALF // IDX_CHUNK):
                c = h * (HALF // IDX_CHUNK) + i
                copies.append(pltpu.async_copy(
                    uf_hbm.at[uidx_v.at[c]],
                    urows_v.at[pl.ds(i * IDX_CHUNK, IDX_CHUNK)], sem))
                copies.append(pltpu.async_copy(
                    bf_hbm.at[bidx_v.at[c]],
                    brows_v.at[pl.ds(i * IDX_CHUNK, IDX_CHUNK)], sem))
            for cp in copies:
                cp.wait()

            def group_body(g, carry, h=h):
                lrow0 = g * L
                ridx = lrow0 + rows16
                goff = h * HALF + lrow0
                ucol = ucol_v[pl.ds(goff, L)]
                bcol = bcol_v[pl.ds(goff, L)]
                upar = upar_v[pl.ds(goff, L)] != zero16
                bpar = bpar_v[pl.ds(goff, L)] != zero16

                def feat_body(f, acc):
                    uw = plsc.load_gather(urows_v, [ridx, ucol + f])
                    bw = plsc.load_gather(brows_v, [ridx, bcol + f])
                    ulo, uhi = plsc.unpack(
                        plsc.bitcast(uw, jnp.bfloat16),
                        format=plsc.PackFormat.INTERLEAVED)
                    blo, bhi = plsc.unpack(
                        plsc.bitcast(bw, jnp.bfloat16),
                        format=plsc.PackFormat.INTERLEAVED)
                    u = jnp.where(upar, uhi, ulo)
                    b = jnp.where(bpar, bhi, blo)
                    return acc + u * b

                acc = lax.fori_loop(0, F, feat_body,
                                    jnp.zeros((L,), jnp.float32), unroll=8)
                out_v[pl.ds(goff, L)] = acc
                return carry

            lax.fori_loop(0, HALF // L, group_body, 0)

        pltpu.sync_copy(out_v, out_hbm.at[pl.ds(base, BPW)])

    return kern


_sc_kernel = _make_sc_kernel()


@jax.jit
def kernel(user, book, user_factors, book_factors):
    user_i = user.astype(jnp.int32)
    book_i = book.astype(jnp.int32)
    uq = user_i // QUART
    bq = book_i // QUART
    user_rec = user_i - uq * QUART
    book_rec = book_i - bq * QUART
    ucol = (uq >> 1) * F
    bcol = (bq >> 1) * F
    upar = uq & 1
    bpar = bq & 1
    uft = user_factors.T
    bft = book_factors.T
    uf2, bf2 = _transpose(uft, uft, uft, uft, bft, bft, bft, bft)
    return _sc_kernel(
        user_rec.reshape(NW, NCHUNK, IDX_CHUNK),
        book_rec.reshape(NW, NCHUNK, IDX_CHUNK),
        ucol, bcol, upar, bpar, uf2, bf2)


# speedup vs baseline: 6.8452x; 1.1921x over previous
import functools

import jax
import jax.numpy as jnp
from jax import lax
from jax.experimental import pallas as pl
from jax.experimental.pallas import tpu as pltpu
from jax.experimental.pallas import tpu_sc as plsc

L = 16
NC = 2
NS = 16
NW = NC * NS
B = 16384
F = 64
N_ROWS = 1000000
QUART = 256000
NREC = QUART
BPW = B // NW
IDX_CHUNK = 128
NCHUNK = BPW // IDX_CHUNK
HALF = BPW // 2

TBLK = 2048
N_CBLK = N_ROWS // TBLK + (1 if N_ROWS % TBLK else 0)
GRID_T = QUART // TBLK
QBLK = QUART // TBLK


def _transpose_body(u0_ref, u1_ref, u2_ref, u3_ref,
                    b0_ref, b1_ref, b2_ref, b3_ref, uo_ref, bo_ref):
    eye = (lax.broadcasted_iota(jnp.int32, (F, F), 0)
           == lax.broadcasted_iota(jnp.int32, (F, F), 1)).astype(jnp.bfloat16)

    def tr(ref):
        return lax.dot_general(ref[...].astype(jnp.bfloat16), eye,
                               (((0,), (0,)), ((), ())),
                               preferred_element_type=jnp.float32)

    def pack2(lo_f32, hi_f32):
        lo = lax.bitcast_convert_type(
            lo_f32.astype(jnp.bfloat16), jnp.uint16).astype(jnp.uint32)
        hi = lax.bitcast_convert_type(
            hi_f32.astype(jnp.bfloat16), jnp.uint16).astype(jnp.uint32)
        return lax.bitcast_convert_type(lo | (hi << 16), jnp.float32)

    uo_ref[...] = jnp.concatenate(
        [pack2(tr(u0_ref), tr(u1_ref)), pack2(tr(u2_ref), tr(u3_ref))],
        axis=1)
    bo_ref[...] = jnp.concatenate(
        [pack2(tr(b0_ref), tr(b1_ref)), pack2(tr(b2_ref), tr(b3_ref))],
        axis=1)


def _window_spec(q):
    return pl.BlockSpec(
        (F, TBLK), lambda g, q=q: (0, jnp.minimum(q * QBLK + g, N_CBLK - 1)))


_transpose = pl.pallas_call(
    _transpose_body,
    grid=(GRID_T,),
    in_specs=[_window_spec(q) for q in range(4)] * 2,
    out_specs=[
        pl.BlockSpec((TBLK, 2 * F), lambda g: (g, 0)),
        pl.BlockSpec((TBLK, 2 * F), lambda g: (g, 0)),
    ],
    out_shape=[
        jax.ShapeDtypeStruct((NREC, 2 * F), jnp.float32),
        jax.ShapeDtypeStruct((NREC, 2 * F), jnp.float32),
    ],
)


def _make_sc_kernel():
    mesh = plsc.VectorSubcoreMesh(core_axis_name="c", subcore_axis_name="s")

    @functools.partial(
        pl.kernel,
        mesh=mesh,
        compiler_params=pltpu.CompilerParams(
            needs_layout_passes=False, use_tc_tiling_on_sc=False),
        out_type=jax.ShapeDtypeStruct((B,), jnp.float32),
        scratch_types=[
            pltpu.VMEM((NCHUNK, IDX_CHUNK), jnp.int32),
            pltpu.VMEM((NCHUNK, IDX_CHUNK), jnp.int32),
            pltpu.VMEM((BPW,), jnp.int32),
            pltpu.VMEM((BPW,), jnp.int32),
            pltpu.VMEM((BPW,), jnp.int32),
            pltpu.VMEM((BPW,), jnp.int32),
            pltpu.VMEM((HALF, 2 * F), jnp.float32),
            pltpu.VMEM((HALF, 2 * F), jnp.float32),
            pltpu.VMEM((BPW,), jnp.float32),
            pltpu.SemaphoreType.DMA,
        ],
    )
    def kern(user_hbm, book_hbm, ucol_hbm, bcol_hbm, upar_hbm, bpar_hbm,
             uf_hbm, bf_hbm, out_hbm,
             uidx_v, bidx_v, ucol_v, bcol_v, upar_v, bpar_v,
             urows_v, brows_v, out_v, sem):
        wid = lax.axis_index("s") * NC + lax.axis_index("c")
        base = wid * BPW

        pltpu.sync_copy(user_hbm.at[wid], uidx_v)
        pltpu.sync_copy(book_hbm.at[wid], bidx_v)
        pltpu.sync_copy(ucol_hbm.at[pl.ds(base, BPW)], ucol_v)
        pltpu.sync_copy(bcol_hbm.at[pl.ds(base, BPW)], bcol_v)
        pltpu.sync_copy(upar_hbm.at[pl.ds(base, BPW)], upar_v)
        pltpu.sync_copy(bpar_hbm.at[pl.ds(base, BPW)], bpar_v)

        rows16 = lax.iota(jnp.int32, L)
        zero16 = jnp.zeros((L,), jnp.int32)

        for h in range(2):
            copies = []
            for i in range(HALF // IDX_CHUNK):
                c = h * (HALF // IDX_CHUNK) + i
                copies.append(pltpu.async_copy(
                    uf_hbm.at[uidx_v.at[c]],
                    urows_v.at[pl.ds(i * IDX_CHUNK, IDX_CHUNK)], sem))
                copies.append(pltpu.async_copy(
                    bf_hbm.at[bidx_v.at[c]],
                    brows_v.at[pl.ds(i * IDX_CHUNK, IDX_CHUNK)], sem))
            for cp in copies:
                cp.wait()

            def group_body(g, carry, h=h):
                lrow0 = g * L
                ridx = lrow0 + rows16
                goff = h * HALF + lrow0
                ucol = ucol_v[pl.ds(goff, L)]
                bcol = bcol_v[pl.ds(goff, L)]
                upar = upar_v[pl.ds(goff, L)] != zero16
                bpar = bpar_v[pl.ds(goff, L)] != zero16

                def feat_body(f, acc):
                    uw = plsc.load_gather(urows_v, [ridx, ucol + f])
                    bw = plsc.load_gather(brows_v, [ridx, bcol + f])
                    ulo, uhi = plsc.unpack(
                        plsc.bitcast(uw, jnp.bfloat16),
                        format=plsc.PackFormat.INTERLEAVED)
                    blo, bhi = plsc.unpack(
                        plsc.bitcast(bw, jnp.bfloat16),
                        format=plsc.PackFormat.INTERLEAVED)
                    u = jnp.where(upar, uhi, ulo)
                    b = jnp.where(bpar, bhi, blo)
                    return acc + u * b

                acc = lax.fori_loop(0, F, feat_body,
                                    jnp.zeros((L,), jnp.float32), unroll=8)
                out_v[pl.ds(goff, L)] = acc
                return carry

            lax.fori_loop(0, HALF // L, group_body, 0)

        pltpu.sync_copy(out_v, out_hbm.at[pl.ds(base, BPW)])

    return kern


_sc_kernel = _make_sc_kernel()


@jax.jit
def kernel(user, book, user_factors, book_factors):
    user_i = user.astype(jnp.int32)
    book_i = book.astype(jnp.int32)
    uq = user_i // QUART
    bq = book_i // QUART
    user_rec = user_i - uq * QUART
    book_rec = book_i - bq * QUART
    ucol = (uq >> 1) * F
    bcol = (bq >> 1) * F
    upar = uq & 1
    bpar = bq & 1
    uft = user_factors.T
    bft = book_factors.T
    uf2, bf2 = _transpose(uft, uft, uft, uft, bft, bft, bft, bft)
    return _sc_kernel(
        user_rec.reshape(NW, NCHUNK, IDX_CHUNK),
        book_rec.reshape(NW, NCHUNK, IDX_CHUNK),
        ucol, bcol, upar, bpar, uf2, bf2)


# speedup vs baseline: 7.1607x; 1.0461x over previous
import functools

import jax
import jax.numpy as jnp
from jax import lax
from jax.experimental import pallas as pl
from jax.experimental.pallas import tpu as pltpu
from jax.experimental.pallas import tpu_sc as plsc

L = 16
NC = 2
NS = 16
NW = NC * NS
B = 16384
F = 64
N_ROWS = 1000000
QUART = 256000
NREC = QUART
BPW = B // NW
IDX_CHUNK = 128
NCHUNK = BPW // IDX_CHUNK
HALF = BPW // 2

TBLK = 2560
N_CBLK = N_ROWS // TBLK + (1 if N_ROWS % TBLK else 0)
GRID_T = QUART // TBLK
QBLK = QUART // TBLK


def _transpose_body(u0_ref, u1_ref, u2_ref, u3_ref,
                    b0_ref, b1_ref, b2_ref, b3_ref, uo_ref, bo_ref):
    eye = (lax.broadcasted_iota(jnp.int32, (F, F), 0)
           == lax.broadcasted_iota(jnp.int32, (F, F), 1)).astype(jnp.bfloat16)

    def tr(ref):
        return lax.dot_general(ref[...].astype(jnp.bfloat16), eye,
                               (((0,), (0,)), ((), ())),
                               preferred_element_type=jnp.float32)

    def pack2(lo_f32, hi_f32):
        lo = lax.bitcast_convert_type(
            lo_f32.astype(jnp.bfloat16), jnp.uint16).astype(jnp.uint32)
        hi = lax.bitcast_convert_type(
            hi_f32.astype(jnp.bfloat16), jnp.uint16).astype(jnp.uint32)
        return lax.bitcast_convert_type(lo | (hi << 16), jnp.float32)

    uo_ref[...] = jnp.concatenate(
        [pack2(tr(u0_ref), tr(u1_ref)), pack2(tr(u2_ref), tr(u3_ref))],
        axis=1)
    bo_ref[...] = jnp.concatenate(
        [pack2(tr(b0_ref), tr(b1_ref)), pack2(tr(b2_ref), tr(b3_ref))],
        axis=1)


def _window_spec(q):
    return pl.BlockSpec(
        (F, TBLK), lambda g, q=q: (0, jnp.minimum(q * QBLK + g, N_CBLK - 1)))


_transpose = pl.pallas_call(
    _transpose_body,
    grid=(GRID_T,),
    in_specs=[_window_spec(q) for q in range(4)] * 2,
    out_specs=[
        pl.BlockSpec((TBLK, 2 * F), lambda g: (g, 0)),
        pl.BlockSpec((TBLK, 2 * F), lambda g: (g, 0)),
    ],
    out_shape=[
        jax.ShapeDtypeStruct((NREC, 2 * F), jnp.float32),
        jax.ShapeDtypeStruct((NREC, 2 * F), jnp.float32),
    ],
)


def _make_sc_kernel():
    mesh = plsc.VectorSubcoreMesh(core_axis_name="c", subcore_axis_name="s")

    @functools.partial(
        pl.kernel,
        mesh=mesh,
        compiler_params=pltpu.CompilerParams(
            needs_layout_passes=False, use_tc_tiling_on_sc=False),
        out_type=jax.ShapeDtypeStruct((B,), jnp.float32),
        scratch_types=[
            pltpu.VMEM((NCHUNK, IDX_CHUNK), jnp.int32),
            pltpu.VMEM((NCHUNK, IDX_CHUNK), jnp.int32),
            pltpu.VMEM((BPW,), jnp.int32),
            pltpu.VMEM((BPW,), jnp.int32),
            pltpu.VMEM((BPW,), jnp.int32),
            pltpu.VMEM((BPW,), jnp.int32),
            pltpu.VMEM((HALF, 2 * F), jnp.float32),
            pltpu.VMEM((HALF, 2 * F), jnp.float32),
            pltpu.VMEM((BPW,), jnp.float32),
            pltpu.SemaphoreType.DMA,
        ],
    )
    def kern(user_hbm, book_hbm, ucol_hbm, bcol_hbm, upar_hbm, bpar_hbm,
             uf_hbm, bf_hbm, out_hbm,
             uidx_v, bidx_v, ucol_v, bcol_v, upar_v, bpar_v,
             urows_v, brows_v, out_v, sem):
        wid = lax.axis_index("s") * NC + lax.axis_index("c")
        base = wid * BPW

        pltpu.sync_copy(user_hbm.at[wid], uidx_v)
        pltpu.sync_copy(book_hbm.at[wid], bidx_v)
        pltpu.sync_copy(ucol_hbm.at[pl.ds(base, BPW)], ucol_v)
        pltpu.sync_copy(bcol_hbm.at[pl.ds(base, BPW)], bcol_v)
        pltpu.sync_copy(upar_hbm.at[pl.ds(base, BPW)], upar_v)
        pltpu.sync_copy(bpar_hbm.at[pl.ds(base, BPW)], bpar_v)

        rows16 = lax.iota(jnp.int32, L)
        zero16 = jnp.zeros((L,), jnp.int32)

        for h in range(2):
            copies = []
            for i in range(HALF // IDX_CHUNK):
                c = h * (HALF // IDX_CHUNK) + i
                copies.append(pltpu.async_copy(
                    uf_hbm.at[uidx_v.at[c]],
                    urows_v.at[pl.ds(i * IDX_CHUNK, IDX_CHUNK)], sem))
                copies.append(pltpu.async_copy(
                    bf_hbm.at[bidx_v.at[c]],
                    brows_v.at[pl.ds(i * IDX_CHUNK, IDX_CHUNK)], sem))
            for cp in copies:
                cp.wait()

            def group_body(g, carry, h=h):
                lrow0 = g * L
                ridx = lrow0 + rows16
                goff = h * HALF + lrow0
                ucol = ucol_v[pl.ds(goff, L)]
                bcol = bcol_v[pl.ds(goff, L)]
                upar = upar_v[pl.ds(goff, L)] != zero16
                bpar = bpar_v[pl.ds(goff, L)] != zero16

                def feat_body(f, acc):
                    uw = plsc.load_gather(urows_v, [ridx, ucol + f])
                    bw = plsc.load_gather(brows_v, [ridx, bcol + f])
                    ulo, uhi = plsc.unpack(
                        plsc.bitcast(uw, jnp.bfloat16),
                        format=plsc.PackFormat.INTERLEAVED)
                    blo, bhi = plsc.unpack(
                        plsc.bitcast(bw, jnp.bfloat16),
                        format=plsc.PackFormat.INTERLEAVED)
                    u = jnp.where(upar, uhi, ulo)
                    b = jnp.where(bpar, bhi, blo)
                    return acc + u * b

                acc = lax.fori_loop(0, F, feat_body,
                                    jnp.zeros((L,), jnp.float32), unroll=8)
                out_v[pl.ds(goff, L)] = acc
                return carry

            lax.fori_loop(0, HALF // L, group_body, 0)

        pltpu.sync_copy(out_v, out_hbm.at[pl.ds(base, BPW)])

    return kern


_sc_kernel = _make_sc_kernel()


@jax.jit
def kernel(user, book, user_factors, book_factors):
    user_i = user.astype(jnp.int32)
    book_i = book.astype(jnp.int32)
    uq = user_i // QUART
    bq = book_i // QUART
    user_rec = user_i - uq * QUART
    book_rec = book_i - bq * QUART
    ucol = (uq >> 1) * F
    bcol = (bq >> 1) * F
    upar = uq & 1
    bpar = bq & 1
    uft = user_factors.T
    bft = book_factors.T
    uf2, bf2 = _transpose(uft, uft, uft, uft, bft, bft, bft, bft)
    return _sc_kernel(
        user_rec.reshape(NW, NCHUNK, IDX_CHUNK),
        book_rec.reshape(NW, NCHUNK, IDX_CHUNK),
        ucol, bcol, upar, bpar, uf2, bf2)


# speedup vs baseline: 7.8720x; 1.0993x over previous
import functools

import jax
import jax.numpy as jnp
from jax import lax
from jax.experimental import pallas as pl
from jax.experimental.pallas import tpu as pltpu
from jax.experimental.pallas import tpu_sc as plsc

L = 16
NC = 2
NS = 16
NW = NC * NS
B = 16384
F = 64
N_ROWS = 1000000
QUART = 256000
NREC = QUART
BPW = B // NW
IDX_CHUNK = 128
NCHUNK = BPW // IDX_CHUNK
HALF = BPW // 2

TBLK = 5120
N_CBLK = N_ROWS // TBLK + (1 if N_ROWS % TBLK else 0)
GRID_T = QUART // TBLK
QBLK = QUART // TBLK


def _transpose_body(u0_ref, u1_ref, u2_ref, u3_ref,
                    b0_ref, b1_ref, b2_ref, b3_ref, uo_ref, bo_ref):
    eye = (lax.broadcasted_iota(jnp.int32, (F, F), 0)
           == lax.broadcasted_iota(jnp.int32, (F, F), 1)).astype(jnp.bfloat16)

    def tr(ref):
        return lax.dot_general(ref[...].astype(jnp.bfloat16), eye,
                               (((0,), (0,)), ((), ())),
                               preferred_element_type=jnp.float32)

    def pack2(lo_f32, hi_f32):
        lo = lax.bitcast_convert_type(
            lo_f32.astype(jnp.bfloat16), jnp.uint16).astype(jnp.uint32)
        hi = lax.bitcast_convert_type(
            hi_f32.astype(jnp.bfloat16), jnp.uint16).astype(jnp.uint32)
        return lax.bitcast_convert_type(lo | (hi << 16), jnp.float32)

    uo_ref[...] = jnp.concatenate(
        [pack2(tr(u0_ref), tr(u1_ref)), pack2(tr(u2_ref), tr(u3_ref))],
        axis=1)
    bo_ref[...] = jnp.concatenate(
        [pack2(tr(b0_ref), tr(b1_ref)), pack2(tr(b2_ref), tr(b3_ref))],
        axis=1)


def _window_spec(q):
    return pl.BlockSpec(
        (F, TBLK), lambda g, q=q: (0, jnp.minimum(q * QBLK + g, N_CBLK - 1)))


_transpose = pl.pallas_call(
    _transpose_body,
    grid=(GRID_T,),
    in_specs=[_window_spec(q) for q in range(4)] * 2,
    out_specs=[
        pl.BlockSpec((TBLK, 2 * F), lambda g: (g, 0)),
        pl.BlockSpec((TBLK, 2 * F), lambda g: (g, 0)),
    ],
    out_shape=[
        jax.ShapeDtypeStruct((NREC, 2 * F), jnp.float32),
        jax.ShapeDtypeStruct((NREC, 2 * F), jnp.float32),
    ],
)


def _make_sc_kernel():
    mesh = plsc.VectorSubcoreMesh(core_axis_name="c", subcore_axis_name="s")

    @functools.partial(
        pl.kernel,
        mesh=mesh,
        compiler_params=pltpu.CompilerParams(
            needs_layout_passes=False, use_tc_tiling_on_sc=False),
        out_type=jax.ShapeDtypeStruct((B,), jnp.float32),
        scratch_types=[
            pltpu.VMEM((NCHUNK, IDX_CHUNK), jnp.int32),
            pltpu.VMEM((NCHUNK, IDX_CHUNK), jnp.int32),
            pltpu.VMEM((BPW,), jnp.int32),
            pltpu.VMEM((BPW,), jnp.int32),
            pltpu.VMEM((BPW,), jnp.int32),
            pltpu.VMEM((BPW,), jnp.int32),
            pltpu.VMEM((HALF, 2 * F), jnp.float32),
            pltpu.VMEM((HALF, 2 * F), jnp.float32),
            pltpu.VMEM((BPW,), jnp.float32),
            pltpu.SemaphoreType.DMA,
        ],
    )
    def kern(user_hbm, book_hbm, ucol_hbm, bcol_hbm, upar_hbm, bpar_hbm,
             uf_hbm, bf_hbm, out_hbm,
             uidx_v, bidx_v, ucol_v, bcol_v, upar_v, bpar_v,
             urows_v, brows_v, out_v, sem):
        wid = lax.axis_index("s") * NC + lax.axis_index("c")
        base = wid * BPW

        pltpu.sync_copy(user_hbm.at[wid], uidx_v)
        pltpu.sync_copy(book_hbm.at[wid], bidx_v)
        pltpu.sync_copy(ucol_hbm.at[pl.ds(base, BPW)], ucol_v)
        pltpu.sync_copy(bcol_hbm.at[pl.ds(base, BPW)], bcol_v)
        pltpu.sync_copy(upar_hbm.at[pl.ds(base, BPW)], upar_v)
        pltpu.sync_copy(bpar_hbm.at[pl.ds(base, BPW)], bpar_v)

        rows16 = lax.iota(jnp.int32, L)
        zero16 = jnp.zeros((L,), jnp.int32)

        for h in range(2):
            copies = []
            for i in range(HALF // IDX_CHUNK):
                c = h * (HALF // IDX_CHUNK) + i
                copies.append(pltpu.async_copy(
                    uf_hbm.at[uidx_v.at[c]],
                    urows_v.at[pl.ds(i * IDX_CHUNK, IDX_CHUNK)], sem))
                copies.append(pltpu.async_copy(
                    bf_hbm.at[bidx_v.at[c]],
                    brows_v.at[pl.ds(i * IDX_CHUNK, IDX_CHUNK)], sem))
            for cp in copies:
                cp.wait()

            def group_body(g, carry, h=h):
                lrow0 = g * L
                ridx = lrow0 + rows16
                goff = h * HALF + lrow0
                ucol = ucol_v[pl.ds(goff, L)]
                bcol = bcol_v[pl.ds(goff, L)]
                upar = upar_v[pl.ds(goff, L)] != zero16
                bpar = bpar_v[pl.ds(goff, L)] != zero16

                def feat_body(f, acc):
                    uw = plsc.load_gather(urows_v, [ridx, ucol + f])
                    bw = plsc.load_gather(brows_v, [ridx, bcol + f])
                    ulo, uhi = plsc.unpack(
                        plsc.bitcast(uw, jnp.bfloat16),
                        format=plsc.PackFormat.INTERLEAVED)
                    blo, bhi = plsc.unpack(
                        plsc.bitcast(bw, jnp.bfloat16),
                        format=plsc.PackFormat.INTERLEAVED)
                    u = jnp.where(upar, uhi, ulo)
                    b = jnp.where(bpar, bhi, blo)
                    return acc + u * b

                acc = lax.fori_loop(0, F, feat_body,
                                    jnp.zeros((L,), jnp.float32), unroll=8)
                out_v[pl.ds(goff, L)] = acc
                return carry

            lax.fori_loop(0, HALF // L, group_body, 0)

        pltpu.sync_copy(out_v, out_hbm.at[pl.ds(base, BPW)])

    return kern


_sc_kernel = _make_sc_kernel()


@jax.jit
def kernel(user, book, user_factors, book_factors):
    user_i = user.astype(jnp.int32)
    book_i = book.astype(jnp.int32)
    uq = user_i // QUART
    bq = book_i // QUART
    user_rec = user_i - uq * QUART
    book_rec = book_i - bq * QUART
    ucol = (uq >> 1) * F
    bcol = (bq >> 1) * F
    upar = uq & 1
    bpar = bq & 1
    uft = user_factors.T
    bft = book_factors.T
    uf2, bf2 = _transpose(uft, uft, uft, uft, bft, bft, bft, bft)
    return _sc_kernel(
        user_rec.reshape(NW, NCHUNK, IDX_CHUNK),
        book_rec.reshape(NW, NCHUNK, IDX_CHUNK),
        ucol, bcol, upar, bpar, uf2, bf2)


# speedup vs baseline: 8.0289x; 1.0199x over previous
import functools

import jax
import jax.numpy as jnp
from jax import lax
from jax.experimental import pallas as pl
from jax.experimental.pallas import tpu as pltpu
from jax.experimental.pallas import tpu_sc as plsc

L = 16
NC = 2
NS = 16
NW = NC * NS
B = 16384
F = 64
N_ROWS = 1000000
QUART = 256000
NREC = QUART
BPW = B // NW
IDX_CHUNK = 128
NCHUNK = BPW // IDX_CHUNK
HALF = BPW // 2

TBLK = 6400
N_CBLK = N_ROWS // TBLK + (1 if N_ROWS % TBLK else 0)
GRID_T = QUART // TBLK
QBLK = QUART // TBLK


def _transpose_body(u0_ref, u1_ref, u2_ref, u3_ref,
                    b0_ref, b1_ref, b2_ref, b3_ref, uo_ref, bo_ref):
    eye = (lax.broadcasted_iota(jnp.int32, (F, F), 0)
           == lax.broadcasted_iota(jnp.int32, (F, F), 1)).astype(jnp.bfloat16)

    def tr(ref):
        return lax.dot_general(ref[...].astype(jnp.bfloat16), eye,
                               (((0,), (0,)), ((), ())),
                               preferred_element_type=jnp.float32)

    def pack2(lo_f32, hi_f32):
        lo = lax.bitcast_convert_type(
            lo_f32.astype(jnp.bfloat16), jnp.uint16).astype(jnp.uint32)
        hi = lax.bitcast_convert_type(
            hi_f32.astype(jnp.bfloat16), jnp.uint16).astype(jnp.uint32)
        return lax.bitcast_convert_type(lo | (hi << 16), jnp.float32)

    uo_ref[...] = jnp.concatenate(
        [pack2(tr(u0_ref), tr(u1_ref)), pack2(tr(u2_ref), tr(u3_ref))],
        axis=1)
    bo_ref[...] = jnp.concatenate(
        [pack2(tr(b0_ref), tr(b1_ref)), pack2(tr(b2_ref), tr(b3_ref))],
        axis=1)


def _window_spec(q):
    return pl.BlockSpec(
        (F, TBLK), lambda g, q=q: (0, jnp.minimum(q * QBLK + g, N_CBLK - 1)))


_transpose = pl.pallas_call(
    _transpose_body,
    grid=(GRID_T,),
    in_specs=[_window_spec(q) for q in range(4)] * 2,
    out_specs=[
        pl.BlockSpec((TBLK, 2 * F), lambda g: (g, 0)),
        pl.BlockSpec((TBLK, 2 * F), lambda g: (g, 0)),
    ],
    out_shape=[
        jax.ShapeDtypeStruct((NREC, 2 * F), jnp.float32),
        jax.ShapeDtypeStruct((NREC, 2 * F), jnp.float32),
    ],
)


def _make_sc_kernel():
    mesh = plsc.VectorSubcoreMesh(core_axis_name="c", subcore_axis_name="s")

    @functools.partial(
        pl.kernel,
        mesh=mesh,
        compiler_params=pltpu.CompilerParams(
            needs_layout_passes=False, use_tc_tiling_on_sc=False),
        out_type=jax.ShapeDtypeStruct((B,), jnp.float32),
        scratch_types=[
            pltpu.VMEM((NCHUNK, IDX_CHUNK), jnp.int32),
            pltpu.VMEM((NCHUNK, IDX_CHUNK), jnp.int32),
            pltpu.VMEM((BPW,), jnp.int32),
            pltpu.VMEM((BPW,), jnp.int32),
            pltpu.VMEM((BPW,), jnp.int32),
            pltpu.VMEM((BPW,), jnp.int32),
            pltpu.VMEM((HALF, 2 * F), jnp.float32),
            pltpu.VMEM((HALF, 2 * F), jnp.float32),
            pltpu.VMEM((BPW,), jnp.float32),
            pltpu.SemaphoreType.DMA,
        ],
    )
    def kern(user_hbm, book_hbm, ucol_hbm, bcol_hbm, upar_hbm, bpar_hbm,
             uf_hbm, bf_hbm, out_hbm,
             uidx_v, bidx_v, ucol_v, bcol_v, upar_v, bpar_v,
             urows_v, brows_v, out_v, sem):
        wid = lax.axis_index("s") * NC + lax.axis_index("c")
        base = wid * BPW

        pltpu.sync_copy(user_hbm.at[wid], uidx_v)
        pltpu.sync_copy(book_hbm.at[wid], bidx_v)
        pltpu.sync_copy(ucol_hbm.at[pl.ds(base, BPW)], ucol_v)
        pltpu.sync_copy(bcol_hbm.at[pl.ds(base, BPW)], bcol_v)
        pltpu.sync_copy(upar_hbm.at[pl.ds(base, BPW)], upar_v)
        pltpu.sync_copy(bpar_hbm.at[pl.ds(base, BPW)], bpar_v)

        rows16 = lax.iota(jnp.int32, L)
        zero16 = jnp.zeros((L,), jnp.int32)

        for h in range(2):
            copies = []
            for i in range(HALF // IDX_CHUNK):
                c = h * (HALF // IDX_CHUNK) + i
                copies.append(pltpu.async_copy(
                    uf_hbm.at[uidx_v.at[c]],
                    urows_v.at[pl.ds(i * IDX_CHUNK, IDX_CHUNK)], sem))
                copies.append(pltpu.async_copy(
                    bf_hbm.at[bidx_v.at[c]],
                    brows_v.at[pl.ds(i * IDX_CHUNK, IDX_CHUNK)], sem))
            for cp in copies:
                cp.wait()

            def group_body(g, carry, h=h):
                lrow0 = g * L
                ridx = lrow0 + rows16
                goff = h * HALF + lrow0
                ucol = ucol_v[pl.ds(goff, L)]
                bcol = bcol_v[pl.ds(goff, L)]
                upar = upar_v[pl.ds(goff, L)] != zero16
                bpar = bpar_v[pl.ds(goff, L)] != zero16

                def feat_body(f, acc):
                    uw = plsc.load_gather(urows_v, [ridx, ucol + f])
                    bw = plsc.load_gather(brows_v, [ridx, bcol + f])
                    ulo, uhi = plsc.unpack(
                        plsc.bitcast(uw, jnp.bfloat16),
                        format=plsc.PackFormat.INTERLEAVED)
                    blo, bhi = plsc.unpack(
                        plsc.bitcast(bw, jnp.bfloat16),
                        format=plsc.PackFormat.INTERLEAVED)
                    u = jnp.where(upar, uhi, ulo)
                    b = jnp.where(bpar, bhi, blo)
                    return acc + u * b

                acc = lax.fori_loop(0, F, feat_body,
                                    jnp.zeros((L,), jnp.float32), unroll=8)
                out_v[pl.ds(goff, L)] = acc
                return carry

            lax.fori_loop(0, HALF // L, group_body, 0)

        pltpu.sync_copy(out_v, out_hbm.at[pl.ds(base, BPW)])

    return kern


_sc_kernel = _make_sc_kernel()


@jax.jit
def kernel(user, book, user_factors, book_factors):
    user_i = user.astype(jnp.int32)
    book_i = book.astype(jnp.int32)
    uq = user_i // QUART
    bq = book_i // QUART
    user_rec = user_i - uq * QUART
    book_rec = book_i - bq * QUART
    ucol = (uq >> 1) * F
    bcol = (bq >> 1) * F
    upar = uq & 1
    bpar = bq & 1
    uft = user_factors.T
    bft = book_factors.T
    uf2, bf2 = _transpose(uft, uft, uft, uft, bft, bft, bft, bft)
    return _sc_kernel(
        user_rec.reshape(NW, NCHUNK, IDX_CHUNK),
        book_rec.reshape(NW, NCHUNK, IDX_CHUNK),
        ucol, bcol, upar, bpar, uf2, bf2)


# speedup vs baseline: 8.1088x; 1.0099x over previous
import functools

import jax
import jax.numpy as jnp
from jax import lax
from jax.experimental import pallas as pl
from jax.experimental.pallas import tpu as pltpu
from jax.experimental.pallas import tpu_sc as plsc

L = 16
NC = 2
NS = 16
NW = NC * NS
B = 16384
F = 64
N_ROWS = 1000000
QUART = 256000
NREC = QUART
BPW = B // NW
IDX_CHUNK = 128
NCHUNK = BPW // IDX_CHUNK
HALF = BPW // 2

TBLK = 6400
N_CBLK = N_ROWS // TBLK + (1 if N_ROWS % TBLK else 0)
GRID_T = QUART // TBLK
QBLK = QUART // TBLK


def _transpose_body(u0_ref, u1_ref, u2_ref, u3_ref,
                    b0_ref, b1_ref, b2_ref, b3_ref, uo_ref, bo_ref):
    eye = (lax.broadcasted_iota(jnp.int32, (F, F), 0)
           == lax.broadcasted_iota(jnp.int32, (F, F), 1)).astype(jnp.bfloat16)

    def tr(ref):
        return lax.dot_general(ref[...].astype(jnp.bfloat16), eye,
                               (((0,), (0,)), ((), ())),
                               preferred_element_type=jnp.float32)

    def pack2(lo_f32, hi_f32):
        lo = lax.bitcast_convert_type(
            lo_f32.astype(jnp.bfloat16), jnp.uint16).astype(jnp.uint32)
        hi = lax.bitcast_convert_type(
            hi_f32.astype(jnp.bfloat16), jnp.uint16).astype(jnp.uint32)
        return lax.bitcast_convert_type(lo | (hi << 16), jnp.float32)

    uo_ref[...] = jnp.concatenate(
        [pack2(tr(u0_ref), tr(u1_ref)), pack2(tr(u2_ref), tr(u3_ref))],
        axis=1)
    bo_ref[...] = jnp.concatenate(
        [pack2(tr(b0_ref), tr(b1_ref)), pack2(tr(b2_ref), tr(b3_ref))],
        axis=1)


def _window_spec(q):
    return pl.BlockSpec(
        (F, TBLK), lambda g, q=q: (0, jnp.minimum(q * QBLK + g, N_CBLK - 1)))


_transpose = pl.pallas_call(
    _transpose_body,
    grid=(GRID_T,),
    in_specs=[_window_spec(q) for q in range(4)] * 2,
    out_specs=[
        pl.BlockSpec((TBLK, 2 * F), lambda g: (g, 0)),
        pl.BlockSpec((TBLK, 2 * F), lambda g: (g, 0)),
    ],
    out_shape=[
        jax.ShapeDtypeStruct((NREC, 2 * F), jnp.float32),
        jax.ShapeDtypeStruct((NREC, 2 * F), jnp.float32),
    ],
)


def _make_sc_kernel():
    mesh = plsc.VectorSubcoreMesh(core_axis_name="c", subcore_axis_name="s")

    @functools.partial(
        pl.kernel,
        mesh=mesh,
        compiler_params=pltpu.CompilerParams(
            needs_layout_passes=False, use_tc_tiling_on_sc=False),
        out_type=jax.ShapeDtypeStruct((B,), jnp.float32),
        scratch_types=[
            pltpu.VMEM((NCHUNK, IDX_CHUNK), jnp.int32),
            pltpu.VMEM((NCHUNK, IDX_CHUNK), jnp.int32),
            pltpu.VMEM((BPW,), jnp.int32),
            pltpu.VMEM((BPW,), jnp.int32),
            pltpu.VMEM((BPW,), jnp.int32),
            pltpu.VMEM((BPW,), jnp.int32),
            pltpu.VMEM((IDX_CHUNK, 2 * F), jnp.float32),
            pltpu.VMEM((IDX_CHUNK, 2 * F), jnp.float32),
            pltpu.VMEM((IDX_CHUNK, 2 * F), jnp.float32),
            pltpu.VMEM((IDX_CHUNK, 2 * F), jnp.float32),
            pltpu.VMEM((BPW,), jnp.float32),
            pltpu.SemaphoreType.DMA,
            pltpu.SemaphoreType.DMA,
        ],
    )
    def kern(user_hbm, book_hbm, ucol_hbm, bcol_hbm, upar_hbm, bpar_hbm,
             uf_hbm, bf_hbm, out_hbm,
             uidx_v, bidx_v, ucol_v, bcol_v, upar_v, bpar_v,
             urows_0, urows_1, brows_0, brows_1, out_v, sem0, sem1):
        wid = lax.axis_index("s") * NC + lax.axis_index("c")
        base = wid * BPW

        pltpu.sync_copy(user_hbm.at[wid], uidx_v)
        pltpu.sync_copy(book_hbm.at[wid], bidx_v)
        pltpu.sync_copy(ucol_hbm.at[pl.ds(base, BPW)], ucol_v)
        pltpu.sync_copy(bcol_hbm.at[pl.ds(base, BPW)], bcol_v)
        pltpu.sync_copy(upar_hbm.at[pl.ds(base, BPW)], upar_v)
        pltpu.sync_copy(bpar_hbm.at[pl.ds(base, BPW)], bpar_v)

        rows16 = lax.iota(jnp.int32, L)
        zero16 = jnp.zeros((L,), jnp.int32)

        ubufs = (urows_0, urows_1)
        bbufs = (brows_0, brows_1)
        sems = (sem0, sem1)

        def fire(q):
            return [
                pltpu.async_copy(uf_hbm.at[uidx_v.at[q]],
                                 ubufs[q % 2], sems[q % 2]),
                pltpu.async_copy(bf_hbm.at[bidx_v.at[q]],
                                 bbufs[q % 2], sems[q % 2]),
            ]

        inflight = fire(0)
        for q in range(NCHUNK):
            nxt = fire(q + 1) if q + 1 < NCHUNK else []
            for cp in inflight:
                cp.wait()
            inflight = nxt
            urows_v = ubufs[q % 2]
            brows_v = bbufs[q % 2]

            def group_body(g, carry, q=q, urows_v=urows_v, brows_v=brows_v):
                lrow0 = g * L
                ridx = lrow0 + rows16
                goff = q * IDX_CHUNK + lrow0
                ucol = ucol_v[pl.ds(goff, L)]
                bcol = bcol_v[pl.ds(goff, L)]
                upar = upar_v[pl.ds(goff, L)] != zero16
                bpar = bpar_v[pl.ds(goff, L)] != zero16

                def feat_body(f, acc):
                    uw = plsc.load_gather(urows_v, [ridx, ucol + f])
                    bw = plsc.load_gather(brows_v, [ridx, bcol + f])
                    ulo, uhi = plsc.unpack(
                        plsc.bitcast(uw, jnp.bfloat16),
                        format=plsc.PackFormat.INTERLEAVED)
                    blo, bhi = plsc.unpack(
                        plsc.bitcast(bw, jnp.bfloat16),
                        format=plsc.PackFormat.INTERLEAVED)
                    u = jnp.where(upar, uhi, ulo)
                    b = jnp.where(bpar, bhi, blo)
                    return acc + u * b

                acc = lax.fori_loop(0, F, feat_body,
                                    jnp.zeros((L,), jnp.float32), unroll=8)
                out_v[pl.ds(goff, L)] = acc
                return carry

            lax.fori_loop(0, IDX_CHUNK // L, group_body, 0)

        pltpu.sync_copy(out_v, out_hbm.at[pl.ds(base, BPW)])

    return kern


_sc_kernel = _make_sc_kernel()


@jax.jit
def kernel(user, book, user_factors, book_factors):
    user_i = user.astype(jnp.int32)
    book_i = book.astype(jnp.int32)
    uq = user_i // QUART
    bq = book_i // QUART
    user_rec = user_i - uq * QUART
    book_rec = book_i - bq * QUART
    ucol = (uq >> 1) * F
    bcol = (bq >> 1) * F
    upar = uq & 1
    bpar = bq & 1
    uft = user_factors.T
    bft = book_factors.T
    uf2, bf2 = _transpose(uft, uft, uft, uft, bft, bft, bft, bft)
    return _sc_kernel(
        user_rec.reshape(NW, NCHUNK, IDX_CHUNK),
        book_rec.reshape(NW, NCHUNK, IDX_CHUNK),
        ucol, bcol, upar, bpar, uf2, bf2)
